# baseline (speedup 1.0000x reference)
import functools

import jax
import jax.numpy as jnp
from jax import lax
from jax.experimental import pallas as pl
from jax.experimental.pallas import tpu as pltpu
from jax.experimental.pallas import tpu_sc as plsc

BK = 256
GCH = 32
CCH = 16


def _router_body(x_ref, wr_ref, br_ref, emb_ref,
                 q_ref, probs_ref, idx_ref, gates_ref):
    x = x_ref[...]
    q = jnp.dot(x, wr_ref[...], preferred_element_type=jnp.float32) + br_ref[...]
    q_ref[...] = q
    emb = emb_ref[...]
    diff = q[:, None, :] - emb[None, :, :]
    scores = -jnp.sum(diff * diff, axis=-1)

    m = jnp.max(scores, axis=-1, keepdims=True)
    ex = jnp.exp(scores - m)
    probs_ref[...] = ex / jnp.sum(ex, axis=-1, keepdims=True)

    ncols = scores.shape[-1]
    col = jax.lax.broadcasted_iota(jnp.int32, scores.shape, 1)
    s1 = jnp.max(scores, axis=-1, keepdims=True)
    a1 = jnp.min(jnp.where(scores == s1, col, ncols), axis=-1, keepdims=True)
    masked = jnp.where(col == a1, -jnp.inf, scores)
    s2 = jnp.max(masked, axis=-1, keepdims=True)
    a2 = jnp.min(jnp.where(masked == s2, col, ncols), axis=-1, keepdims=True)

    e21 = jnp.exp(s2 - s1)
    g1 = 1.0 / (1.0 + e21)
    g2 = e21 / (1.0 + e21)
    idx_ref[...] = jnp.concatenate([a1, a2], axis=-1)
    gates_ref[...] = jnp.concatenate([g1, g2], axis=-1)


def _router(x, Wr, br, expert_emb, tb):
    B, D = x.shape
    EMB = Wr.shape[1]
    E = expert_emb.shape[0]
    nt = B // tb
    return pl.pallas_call(
        _router_body,
        grid=(nt,),
        in_specs=[
            pl.BlockSpec((tb, D), lambda t: (t, 0)),
            pl.BlockSpec((D, EMB), lambda t: (0, 0)),
            pl.BlockSpec((EMB,), lambda t: (0,)),
            pl.BlockSpec((E, EMB), lambda t: (0, 0)),
        ],
        out_specs=[
            pl.BlockSpec((tb, EMB), lambda t: (t, 0)),
            pl.BlockSpec((tb, E), lambda t: (t, 0)),
            pl.BlockSpec((tb, 2), lambda t: (t, 0)),
            pl.BlockSpec((tb, 2), lambda t: (t, 0)),
        ],
        out_shape=[
            jax.ShapeDtypeStruct((B, EMB), jnp.float32),
            jax.ShapeDtypeStruct((B, E), jnp.float32),
            jax.ShapeDtypeStruct((B, 2), jnp.int32),
            jax.ShapeDtypeStruct((B, 2), jnp.float32),
        ],
    )(x, Wr, br, expert_emb)


def _dispatch_plan(idx2, gates2, E, padded):
    B = idx2.shape[0]
    flat_e = idx2.reshape(-1)
    onehot = (flat_e[:, None] == jnp.arange(E, dtype=jnp.int32)[None, :])
    counts = jnp.sum(onehot.astype(jnp.int32), axis=0)
    ranks = jnp.cumsum(onehot.astype(jnp.int32), axis=0)
    rank = jnp.sum(jnp.where(onehot, ranks, 0), axis=1) - 1
    padded_counts = ((counts + BK - 1) // BK) * BK
    ends = jnp.cumsum(padded_counts)
    offs = ends - padded_counts
    pos = offs[flat_e] + rank

    nb = padded // BK
    first_row = jnp.arange(nb, dtype=jnp.int32) * BK
    block_expert = jnp.minimum(
        jnp.sum((first_row[:, None] >= ends[None, :]).astype(jnp.int32), axis=1),
        E - 1).astype(jnp.int32)

    tok = jnp.zeros((padded,), jnp.int32).at[pos].set(
        jnp.arange(2 * B, dtype=jnp.int32) // 2)
    gates_sorted = jnp.zeros((padded,), jnp.float32).at[pos].set(
        gates2.reshape(-1))
    pos2 = pos.reshape(B, 2)
    return tok, gates_sorted, block_expert, pos2[:, 0], pos2[:, 1]


def _sc_gather(x, tok, padded):
    B, D = x.shape
    info = plsc.get_sparse_core_info()
    nw = info.num_cores * info.num_subcores
    rows_per_w = padded // nw
    nch = rows_per_w // GCH
    mesh = plsc.VectorSubcoreMesh(core_axis_name="c", subcore_axis_name="s")

    @functools.partial(
        pl.kernel, mesh=mesh,
        out_type=jax.ShapeDtypeStruct((padded, D), jnp.float32),
        scratch_types=[
            pltpu.VMEM((GCH,), jnp.int32),
            pltpu.VMEM((GCH, D), jnp.float32),
            pltpu.SemaphoreType.DMA,
        ],
    )
    def k(x_hbm, tok_hbm, xs_hbm, idx_v, rows_v, sem):
        wid = lax.axis_index("s") * info.num_cores + lax.axis_index("c")
        base = wid * rows_per_w
        for c in range(nch):
            off = base + c * GCH
            pltpu.sync_copy(tok_hbm.at[pl.ds(off, GCH)], idx_v)
            pltpu.async_copy(x_hbm.at[idx_v], rows_v, sem).wait()
            pltpu.sync_copy(rows_v, xs_hbm.at[pl.ds(off, GCH)])

    return k(x, tok)


def _sc_combine(ys, p1, p2):
    B = p1.shape[0]
    D = ys.shape[1]
    info = plsc.get_sparse_core_info()
    nw = info.num_cores * info.num_subcores
    rows_per_w = B // nw
    nch = rows_per_w // CCH
    nvec = D // 16
    mesh = plsc.VectorSubcoreMesh(core_axis_name="c", subcore_axis_name="s")

    @functools.partial(
        pl.kernel, mesh=mesh,
        out_type=jax.ShapeDtypeStruct((B, D), jnp.float32),
        scratch_types=[
            pltpu.VMEM((CCH,), jnp.int32),
            pltpu.VMEM((CCH,), jnp.int32),
            pltpu.VMEM((CCH, D), jnp.float32),
            pltpu.VMEM((CCH, D), jnp.float32),
            pltpu.SemaphoreType.DMA,
        ],
    )
    def k(ys_hbm, p1_hbm, p2_hbm, out_hbm, i1_v, i2_v, a_v, b_v, sem):
        wid = lax.axis_index("s") * info.num_cores + lax.axis_index("c")
        base = wid * rows_per_w
        for c in range(nch):
            off = base + c * CCH
            pltpu.sync_copy(p1_hbm.at[pl.ds(off, CCH)], i1_v)
            pltpu.sync_copy(p2_hbm.at[pl.ds(off, CCH)], i2_v)
            pltpu.async_copy(ys_hbm.at[i1_v], a_v, sem).wait()
            pltpu.async_copy(ys_hbm.at[i2_v], b_v, sem).wait()
            for r in range(CCH):
                def body(j, _, r=r):
                    sl = pl.ds(j * 16, 16)
                    a_v[r, sl] = a_v[r, sl] + b_v[r, sl]
                    return 0
                lax.fori_loop(0, nvec, body, 0, unroll=4)
            pltpu.sync_copy(a_v, out_hbm.at[pl.ds(off, CCH)])

    return k(ys, p1, p2)


def _experts_body(be_ref, xs_ref, w1_ref, b1_ref, w2_ref, b2_ref,
                  w3_ref, b3_ref, g_ref, out_ref):
    x = xs_ref[...]
    h1 = jnp.maximum(
        jnp.dot(x, w1_ref[0], preferred_element_type=jnp.float32) + b1_ref[0], 0.0)
    h2 = jnp.maximum(
        jnp.dot(h1, w2_ref[0], preferred_element_type=jnp.float32) + b2_ref[0], 0.0)
    y = jnp.dot(h2, w3_ref[0], preferred_element_type=jnp.float32) + b3_ref[0]
    out_ref[...] = y * g_ref[...]


def _experts(xs, W1, b1, W2, b2, W3, b3, gates_sorted, block_expert):
    padded, D = xs.shape
    E, _, H = W1.shape
    Ho = W2.shape[2]
    C = W3.shape[2]
    nb = padded // BK
    grid_spec = pltpu.PrefetchScalarGridSpec(
        num_scalar_prefetch=1,
        grid=(nb,),
        in_specs=[
            pl.BlockSpec((BK, D), lambda i, be: (i, 0)),
            pl.BlockSpec((1, D, H), lambda i, be: (be[i], 0, 0)),
            pl.BlockSpec((1, 1, H), lambda i, be: (be[i], 0, 0)),
            pl.BlockSpec((1, H, Ho), lambda i, be: (be[i], 0, 0)),
            pl.BlockSpec((1, 1, Ho), lambda i, be: (be[i], 0, 0)),
            pl.BlockSpec((1, Ho, C), lambda i, be: (be[i], 0, 0)),
            pl.BlockSpec((1, 1, C), lambda i, be: (be[i], 0, 0)),
            pl.BlockSpec((BK, 1), lambda i, be: (i, 0)),
        ],
        out_specs=pl.BlockSpec((BK, C), lambda i, be: (i, 0)),
    )
    return pl.pallas_call(
        _experts_body,
        grid_spec=grid_spec,
        out_shape=jax.ShapeDtypeStruct((padded, C), jnp.float32),
    )(block_expert, xs, W1, b1[:, None, :], W2, b2[:, None, :],
      W3, b3[:, None, :], gates_sorted[:, None])


def kernel(inputs, Wr, br, expert_emb, W1, b1, W2, b2, W3, b3):
    B = inputs.shape[0]
    E = expert_emb.shape[0]
    padded = 2 * B + E * BK
    tb = 512 if B % 512 == 0 else B
    query, probs, idx2, gates2 = _router(inputs, Wr, br, expert_emb, tb)
    tok, gates_sorted, block_expert, p1, p2 = _dispatch_plan(idx2, gates2, E, padded)
    xs = _sc_gather(inputs, tok, padded)
    ys = _experts(xs, W1, b1, W2, b2, W3, b3, gates_sorted, block_expert)
    combined = _sc_combine(ys, p1, p2)
    return combined, query, probs


# speedup vs baseline: 1.4998x; 1.4998x over previous
import functools

import jax
import jax.numpy as jnp
from jax import lax
from jax.experimental import pallas as pl
from jax.experimental.pallas import tpu as pltpu
from jax.experimental.pallas import tpu_sc as plsc

BK = 256
GCH = 32
CCH = 16


def _router_body(x_ref, wr_ref, br_ref, emb_ref,
                 q_ref, probs_ref, idx_ref, gates_ref):
    x = x_ref[...]
    q = jnp.dot(x, wr_ref[...], preferred_element_type=jnp.float32) + br_ref[...]
    q_ref[...] = q
    emb = emb_ref[...]
    diff = q[:, None, :] - emb[None, :, :]
    scores = -jnp.sum(diff * diff, axis=-1)

    m = jnp.max(scores, axis=-1, keepdims=True)
    ex = jnp.exp(scores - m)
    probs_ref[...] = ex / jnp.sum(ex, axis=-1, keepdims=True)

    ncols = scores.shape[-1]
    col = jax.lax.broadcasted_iota(jnp.int32, scores.shape, 1)
    s1 = jnp.max(scores, axis=-1, keepdims=True)
    a1 = jnp.min(jnp.where(scores == s1, col, ncols), axis=-1, keepdims=True)
    masked = jnp.where(col == a1, -jnp.inf, scores)
    s2 = jnp.max(masked, axis=-1, keepdims=True)
    a2 = jnp.min(jnp.where(masked == s2, col, ncols), axis=-1, keepdims=True)

    e21 = jnp.exp(s2 - s1)
    g1 = 1.0 / (1.0 + e21)
    g2 = e21 / (1.0 + e21)
    idx_ref[...] = jnp.concatenate([a1, a2], axis=-1)
    gates_ref[...] = jnp.concatenate([g1, g2], axis=-1)


def _router(x, Wr, br, expert_emb, tb):
    B, D = x.shape
    EMB = Wr.shape[1]
    E = expert_emb.shape[0]
    nt = B // tb
    return pl.pallas_call(
        _router_body,
        grid=(nt,),
        in_specs=[
            pl.BlockSpec((tb, D), lambda t: (t, 0)),
            pl.BlockSpec((D, EMB), lambda t: (0, 0)),
            pl.BlockSpec((EMB,), lambda t: (0,)),
            pl.BlockSpec((E, EMB), lambda t: (0, 0)),
        ],
        out_specs=[
            pl.BlockSpec((tb, EMB), lambda t: (t, 0)),
            pl.BlockSpec((tb, E), lambda t: (t, 0)),
            pl.BlockSpec((tb, 2), lambda t: (t, 0)),
            pl.BlockSpec((tb, 2), lambda t: (t, 0)),
        ],
        out_shape=[
            jax.ShapeDtypeStruct((B, EMB), jnp.float32),
            jax.ShapeDtypeStruct((B, E), jnp.float32),
            jax.ShapeDtypeStruct((B, 2), jnp.int32),
            jax.ShapeDtypeStruct((B, 2), jnp.float32),
        ],
    )(x, Wr, br, expert_emb)


def _dispatch_plan(idx2, E, padded):
    B = idx2.shape[0]
    flat_e = idx2.reshape(-1)
    onehot = (flat_e[:, None] == jnp.arange(E, dtype=jnp.int32)[None, :])
    counts = jnp.sum(onehot.astype(jnp.int32), axis=0)
    ranks = jnp.cumsum(onehot.astype(jnp.int32), axis=0)
    rank = jnp.sum(jnp.where(onehot, ranks, 0), axis=1) - 1
    padded_counts = ((counts + BK - 1) // BK) * BK
    ends = jnp.cumsum(padded_counts)
    offs = ends - padded_counts
    pos = offs[flat_e] + rank

    nb = padded // BK
    first_row = jnp.arange(nb, dtype=jnp.int32) * BK
    block_expert = jnp.minimum(
        jnp.sum((first_row[:, None] >= ends[None, :]).astype(jnp.int32), axis=1),
        E - 1).astype(jnp.int32)

    pos2 = pos.reshape(B, 2)
    return pos, block_expert, pos2[:, 0], pos2[:, 1]


def _sc_dispatch(x, p1, p2, pos, gflat, padded):
    B, D = x.shape
    info = plsc.get_sparse_core_info()
    nw = info.num_cores * info.num_subcores
    tpw = B // nw
    ch = min(GCH, tpw)
    nch = tpw // ch
    mesh = plsc.VectorSubcoreMesh(core_axis_name="c", subcore_axis_name="s")

    buf_t = [
        pltpu.VMEM((ch, D), jnp.float32),
        pltpu.VMEM((ch,), jnp.int32),
        pltpu.VMEM((ch,), jnp.int32),
        pltpu.VMEM((2 * ch,), jnp.int32),
        pltpu.VMEM((2 * ch,), jnp.float32),
        pltpu.SemaphoreType.DMA,
        pltpu.SemaphoreType.DMA,
    ]

    @functools.partial(
        pl.kernel, mesh=mesh,
        out_type=[
            jax.ShapeDtypeStruct((padded, D), jnp.float32),
            jax.ShapeDtypeStruct((padded,), jnp.float32),
        ],
        scratch_types=buf_t + buf_t,
    )
    def k(x_hbm, p1_hbm, p2_hbm, pos_hbm, g_hbm, xs_hbm, gs_hbm, *scr):
        sets = [scr[:7], scr[7:]]
        wid = lax.axis_index("s") * info.num_cores + lax.axis_index("c")
        base = wid * tpw
        ld, st = {}, {}

        def start_load(c):
            xb, i1, i2, ip, gb, sl, _ = sets[c % 2]
            o = base + c * ch
            ld[c] = [
                pltpu.async_copy(x_hbm.at[pl.ds(o, ch)], xb, sl),
                pltpu.async_copy(p1_hbm.at[pl.ds(o, ch)], i1, sl),
                pltpu.async_copy(p2_hbm.at[pl.ds(o, ch)], i2, sl),
                pltpu.async_copy(pos_hbm.at[pl.ds(2 * o, 2 * ch)], ip, sl),
                pltpu.async_copy(g_hbm.at[pl.ds(2 * o, 2 * ch)], gb, sl),
            ]

        def start_store(c):
            xb, i1, i2, ip, gb, _, ss = sets[c % 2]
            st[c] = [
                pltpu.async_copy(xb, xs_hbm.at[i1], ss),
                pltpu.async_copy(xb, xs_hbm.at[i2], ss),
                pltpu.async_copy(gb, gs_hbm.at[ip], ss),
            ]

        start_load(0)
        for c in range(nch):
            if c + 1 < nch:
                if c - 1 >= 0:
                    for d in st.pop(c - 1):
                        d.wait()
                start_load(c + 1)
            for d in ld.pop(c):
                d.wait()
            start_store(c)
        for c in sorted(st):
            for d in st.pop(c):
                d.wait()

    return k(x, p1, p2, pos, gflat)


def _sc_combine(ys, p1, p2):
    B = p1.shape[0]
    D = ys.shape[1]
    info = plsc.get_sparse_core_info()
    nw = info.num_cores * info.num_subcores
    tpw = B // nw
    ch = min(CCH, tpw)
    nch = tpw // ch
    nvec = D // 16
    mesh = plsc.VectorSubcoreMesh(core_axis_name="c", subcore_axis_name="s")

    buf_t = [
        pltpu.VMEM((ch,), jnp.int32),
        pltpu.VMEM((ch,), jnp.int32),
        pltpu.VMEM((ch, D), jnp.float32),
        pltpu.VMEM((ch, D), jnp.float32),
        pltpu.SemaphoreType.DMA,
        pltpu.SemaphoreType.DMA,
    ]

    @functools.partial(
        pl.kernel, mesh=mesh,
        out_type=jax.ShapeDtypeStruct((B, D), jnp.float32),
        scratch_types=buf_t + buf_t,
    )
    def k(ys_hbm, p1_hbm, p2_hbm, out_hbm, *scr):
        sets = [scr[:6], scr[6:]]
        wid = lax.axis_index("s") * info.num_cores + lax.axis_index("c")
        base = wid * tpw
        ld, st = {}, {}

        def start_load(c):
            i1, i2, a, b, sl, _ = sets[c % 2]
            o = base + c * ch
            pltpu.sync_copy(p1_hbm.at[pl.ds(o, ch)], i1)
            pltpu.sync_copy(p2_hbm.at[pl.ds(o, ch)], i2)
            ld[c] = [
                pltpu.async_copy(ys_hbm.at[i1], a, sl),
                pltpu.async_copy(ys_hbm.at[i2], b, sl),
            ]

        def start_store(c):
            i1, i2, a, b, _, ss = sets[c % 2]
            o = base + c * ch
            st[c] = [pltpu.async_copy(a, out_hbm.at[pl.ds(o, ch)], ss)]

        start_load(0)
        for c in range(nch):
            if c + 1 < nch:
                if c - 1 >= 0:
                    for d in st.pop(c - 1):
                        d.wait()
                start_load(c + 1)
            for d in ld.pop(c):
                d.wait()
            _, _, a, b, _, _ = sets[c % 2]
            for r in range(ch):
                def body(j, _, r=r):
                    sl = pl.ds(j * 16, 16)
                    a[r, sl] = a[r, sl] + b[r, sl]
                    return 0
                lax.fori_loop(0, nvec, body, 0, unroll=4)
            start_store(c)
        for c in sorted(st):
            for d in st.pop(c):
                d.wait()

    return k(ys, p1, p2)


def _experts_body(be_ref, xs_ref, w1_ref, b1_ref, w2_ref, b2_ref,
                  w3_ref, b3_ref, g_ref, out_ref):
    x = xs_ref[...]
    h1 = jnp.maximum(
        jnp.dot(x, w1_ref[0], preferred_element_type=jnp.float32) + b1_ref[0], 0.0)
    h2 = jnp.maximum(
        jnp.dot(h1, w2_ref[0], preferred_element_type=jnp.float32) + b2_ref[0], 0.0)
    y = jnp.dot(h2, w3_ref[0], preferred_element_type=jnp.float32) + b3_ref[0]
    out_ref[...] = y * g_ref[...]


def _experts(xs, W1, b1, W2, b2, W3, b3, gates_sorted, block_expert):
    padded, D = xs.shape
    E, _, H = W1.shape
    Ho = W2.shape[2]
    C = W3.shape[2]
    nb = padded // BK
    grid_spec = pltpu.PrefetchScalarGridSpec(
        num_scalar_prefetch=1,
        grid=(nb,),
        in_specs=[
            pl.BlockSpec((BK, D), lambda i, be: (i, 0)),
            pl.BlockSpec((1, D, H), lambda i, be: (be[i], 0, 0)),
            pl.BlockSpec((1, 1, H), lambda i, be: (be[i], 0, 0)),
            pl.BlockSpec((1, H, Ho), lambda i, be: (be[i], 0, 0)),
            pl.BlockSpec((1, 1, Ho), lambda i, be: (be[i], 0, 0)),
            pl.BlockSpec((1, Ho, C), lambda i, be: (be[i], 0, 0)),
            pl.BlockSpec((1, 1, C), lambda i, be: (be[i], 0, 0)),
            pl.BlockSpec((BK, 1), lambda i, be: (i, 0)),
        ],
        out_specs=pl.BlockSpec((BK, C), lambda i, be: (i, 0)),
    )
    return pl.pallas_call(
        _experts_body,
        grid_spec=grid_spec,
        out_shape=jax.ShapeDtypeStruct((padded, C), jnp.float32),
    )(block_expert, xs, W1, b1[:, None, :], W2, b2[:, None, :],
      W3, b3[:, None, :], gates_sorted[:, None])


def kernel(inputs, Wr, br, expert_emb, W1, b1, W2, b2, W3, b3):
    B = inputs.shape[0]
    E = expert_emb.shape[0]
    padded = 2 * B + E * BK
    tb = 512 if B % 512 == 0 else B
    query, probs, idx2, gates2 = _router(inputs, Wr, br, expert_emb, tb)
    pos, block_expert, p1, p2 = _dispatch_plan(idx2, E, padded)
    xs, gs = _sc_dispatch(inputs, p1, p2, pos, gates2.reshape(-1), padded)
    ys = _experts(xs, W1, b1, W2, b2, W3, b3, gs, block_expert)
    combined = _sc_combine(ys, p1, p2)
    return combined, query, probs
